# Initial kernel scaffold; baseline (speedup 1.0000x reference)
import jax
import jax.numpy as jnp
from jax.experimental import pallas as pl


def kernel(x, W_ctl, b_ctl, W_exp, b_exp):
    raise NotImplementedError("write your pallas kernel here")


# speedup vs baseline: 2.1327x; 2.1327x over previous
import functools

import jax
import jax.numpy as jnp
from jax import lax
from jax.experimental import pallas as pl
from jax.experimental.pallas import tpu as pltpu
from jax.experimental.pallas import tpu_sc as plsc

B, C_IN, C_OUT, H, W_DIM, E, K = 16, 384, 384, 56, 56, 8, 3
HP, WP = H + 2, W_DIM + 2
NPIX = HP * WP
M_TILE = 3368
PAD = 60
XROWS = PAD + NPIX + (M_TILE - NPIX) + 59 + 1


def _router_body(x_ref, wc_ref, bc_ref, dec_ref):
    s = jnp.sum(x_ref[0].reshape(C_IN, H * W_DIM), axis=1)
    mean = s * (1.0 / (H * W_DIM))
    logits = jnp.sum(wc_ref[...] * mean[None, :], axis=1) + bc_ref[...]
    maxv = jnp.max(logits)
    idx = lax.broadcasted_iota(jnp.int32, (E,), 0)
    dec = jnp.min(jnp.where(logits == maxv, idx, E + 1))
    dec_ref[pl.program_id(0)] = dec


def _conv_body(sd_ref, od_ref, x_ref, w_ref, b_ref, o_ref):
    acc = jnp.zeros((M_TILE, C_OUT), dtype=jnp.float32)
    for k in range(9):
        kh, kw = k // 3, k % 3
        s = (kh - 1) * WP + (kw - 1)
        xs = x_ref[0, pl.ds(PAD + s, M_TILE), :]
        acc = acc + lax.dot_general(
            xs, w_ref[0, k],
            (((1,), (0,)), ((), ())),
            preferred_element_type=jnp.float32,
        )
    o_ref[0] = acc + b_ref[0, 0, :]


def _sc_dispatch(dec_hbm, sdec_hbm, ord_hbm, dec_v, sdec_v, ord_v):
    wid = lax.axis_index("s") * 2 + lax.axis_index("c")

    @pl.when(wid == 0)
    def _():
        pltpu.sync_copy(dec_hbm, dec_v)
        keys = dec_v[...]
        lanes = lax.iota(jnp.int32, B)
        sd, so = plsc.sort_key_val(keys, lanes)
        sdec_v[...] = sd
        ord_v[...] = so
        pltpu.sync_copy(sdec_v, sdec_hbm)
        pltpu.sync_copy(ord_v, ord_hbm)


def kernel(x, W_ctl, b_ctl, W_exp, b_exp):
    decisions = pl.pallas_call(
        _router_body,
        grid=(B,),
        in_specs=[
            pl.BlockSpec((1, C_IN, H, W_DIM), lambda b: (b, 0, 0, 0)),
            pl.BlockSpec((E, C_IN), lambda b: (0, 0)),
            pl.BlockSpec((E,), lambda b: (0,)),
        ],
        out_specs=pl.BlockSpec((B,), lambda b: (0,), memory_space=pltpu.SMEM),
        out_shape=jax.ShapeDtypeStruct((B,), jnp.int32),
    )(x, W_ctl[:, :, 0, 0], b_ctl)

    mesh = plsc.VectorSubcoreMesh(core_axis_name="c", subcore_axis_name="s")
    sorted_dec, order = pl.kernel(
        _sc_dispatch,
        mesh=mesh,
        out_type=[
            jax.ShapeDtypeStruct((B,), jnp.int32),
            jax.ShapeDtypeStruct((B,), jnp.int32),
        ],
        scratch_types=[
            pltpu.VMEM((B,), jnp.int32),
            pltpu.VMEM((B,), jnp.int32),
            pltpu.VMEM((B,), jnp.int32),
        ],
        compiler_params=pltpu.CompilerParams(needs_layout_passes=False),
    )(decisions)

    x_t = jnp.pad(x.transpose(0, 2, 3, 1), ((0, 0), (1, 1), (1, 1), (0, 0)))
    x_t = x_t.reshape(B, NPIX, C_IN)
    x_t = jnp.pad(x_t, ((0, 0), (PAD, XROWS - PAD - NPIX), (0, 0)))
    w_r = W_exp.transpose(0, 3, 4, 2, 1).reshape(E, 9, C_IN, C_OUT)

    grid_spec = pltpu.PrefetchScalarGridSpec(
        num_scalar_prefetch=2,
        grid=(B,),
        in_specs=[
            pl.BlockSpec((1, XROWS, C_IN), lambda b, sd, od: (od[b], 0, 0)),
            pl.BlockSpec((1, 9, C_IN, C_OUT), lambda b, sd, od: (sd[b], 0, 0, 0)),
            pl.BlockSpec((1, 1, C_OUT), lambda b, sd, od: (sd[b], 0, 0)),
        ],
        out_specs=pl.BlockSpec((1, M_TILE, C_OUT), lambda b, sd, od: (od[b], 0, 0)),
    )
    out = pl.pallas_call(
        _conv_body,
        grid_spec=grid_spec,
        out_shape=jax.ShapeDtypeStruct((B, M_TILE, C_OUT), jnp.float32),
    )(sorted_dec, order, x_t, w_r, b_exp.reshape(E, 1, C_OUT))

    y = out[:, :NPIX, :].reshape(B, HP, WP, C_OUT)
    return y[:, 1:-1, 1:-1, :].transpose(0, 3, 1, 2)


# speedup vs baseline: 2.3607x; 1.1069x over previous
import functools

import jax
import jax.numpy as jnp
from jax import lax
from jax.experimental import pallas as pl
from jax.experimental.pallas import tpu as pltpu
from jax.experimental.pallas import tpu_sc as plsc

B, C_IN, C_OUT, H, W_DIM, E, K = 16, 384, 384, 56, 56, 8, 3
HP, WP = H + 2, W_DIM + 2
NPIX = HP * WP
M_TILE = 3368
PAD = 60
XROWS = PAD + NPIX + (M_TILE - NPIX) + 59 + 1


def _router_body(x_ref, wc_ref, bc_ref, dec_ref):
    s = jnp.sum(x_ref[0].reshape(C_IN, H * W_DIM), axis=1)
    mean = s * (1.0 / (H * W_DIM))
    logits = jnp.sum(wc_ref[...] * mean[None, :], axis=1) + bc_ref[...]
    maxv = jnp.max(logits)
    idx = lax.broadcasted_iota(jnp.int32, (E,), 0)
    dec = jnp.min(jnp.where(logits == maxv, idx, E + 1))
    dec_ref[pl.program_id(0)] = dec


def _conv_body(sd_ref, od_ref, x_ref, w_ref, b_ref, o_ref):
    acc = jnp.zeros((M_TILE, C_OUT), dtype=jnp.float32)
    for k in range(9):
        kh, kw = k // 3, k % 3
        s = (kh - 1) * WP + (kw - 1)
        xs = x_ref[0, pl.ds(PAD + s, M_TILE), :]
        acc = acc + lax.dot_general(
            xs, w_ref[0, k],
            (((1,), (0,)), ((), ())),
            preferred_element_type=jnp.float32,
        )
    o_ref[0] = acc + b_ref[0, 0, :]


def _sc_dispatch(dec_hbm, sdec_hbm, ord_hbm, dec_v, sdec_v, ord_v):
    wid = lax.axis_index("s") * 2 + lax.axis_index("c")

    @pl.when(wid == 0)
    def _():
        pltpu.sync_copy(dec_hbm, dec_v)
        keys = dec_v[...]
        lanes = lax.iota(jnp.int32, B)
        sd, so = plsc.sort_key_val(keys, lanes)
        sdec_v[...] = sd
        ord_v[...] = so
        pltpu.sync_copy(sdec_v, sdec_hbm)
        pltpu.sync_copy(ord_v, ord_hbm)


def kernel(x, W_ctl, b_ctl, W_exp, b_exp):
    decisions = pl.pallas_call(
        _router_body,
        grid=(B,),
        in_specs=[
            pl.BlockSpec((1, C_IN, H, W_DIM), lambda b: (b, 0, 0, 0)),
            pl.BlockSpec((E, C_IN), lambda b: (0, 0)),
            pl.BlockSpec((E,), lambda b: (0,)),
        ],
        out_specs=pl.BlockSpec((B,), lambda b: (0,), memory_space=pltpu.SMEM),
        out_shape=jax.ShapeDtypeStruct((B,), jnp.int32),
    )(x, W_ctl[:, :, 0, 0], b_ctl)

    mesh = plsc.VectorSubcoreMesh(core_axis_name="c", subcore_axis_name="s")
    sorted_dec, order = pl.kernel(
        _sc_dispatch,
        mesh=mesh,
        out_type=[
            jax.ShapeDtypeStruct((B,), jnp.int32),
            jax.ShapeDtypeStruct((B,), jnp.int32),
        ],
        scratch_types=[
            pltpu.VMEM((B,), jnp.int32),
            pltpu.VMEM((B,), jnp.int32),
            pltpu.VMEM((B,), jnp.int32),
        ],
        compiler_params=pltpu.CompilerParams(needs_layout_passes=False),
    )(decisions)

    x_bf = x.astype(jnp.bfloat16)
    x_t = jnp.pad(x_bf.transpose(0, 2, 3, 1), ((0, 0), (1, 1), (1, 1), (0, 0)))
    x_t = x_t.reshape(B, NPIX, C_IN)
    x_t = jnp.pad(x_t, ((0, 0), (PAD, XROWS - PAD - NPIX), (0, 0)))
    w_r = W_exp.astype(jnp.bfloat16).transpose(0, 3, 4, 2, 1).reshape(E, 9, C_IN, C_OUT)

    grid_spec = pltpu.PrefetchScalarGridSpec(
        num_scalar_prefetch=2,
        grid=(B,),
        in_specs=[
            pl.BlockSpec((1, XROWS, C_IN), lambda b, sd, od: (od[b], 0, 0)),
            pl.BlockSpec((1, 9, C_IN, C_OUT), lambda b, sd, od: (sd[b], 0, 0, 0)),
            pl.BlockSpec((1, 1, C_OUT), lambda b, sd, od: (sd[b], 0, 0)),
        ],
        out_specs=pl.BlockSpec((1, M_TILE, C_OUT), lambda b, sd, od: (od[b], 0, 0)),
    )
    out = pl.pallas_call(
        _conv_body,
        grid_spec=grid_spec,
        out_shape=jax.ShapeDtypeStruct((B, M_TILE, C_OUT), jnp.float32),
    )(sorted_dec, order, x_t, w_r, b_exp.reshape(E, 1, C_OUT))

    y = out[:, :NPIX, :].reshape(B, HP, WP, C_OUT)
    return y[:, 1:-1, 1:-1, :].transpose(0, 3, 1, 2)


# speedup vs baseline: 3.2051x; 1.3577x over previous
import functools

import jax
import jax.numpy as jnp
from jax import lax
from jax.experimental import pallas as pl
from jax.experimental.pallas import tpu as pltpu
from jax.experimental.pallas import tpu_sc as plsc

B, C_IN, C_OUT, H, W_DIM, E, K = 16, 384, 384, 56, 56, 8, 3
NPIX = H * W_DIM
LPAD = 64
XCOLS = NPIX + 2 * LPAD


def _router_body(x_ref, wc_ref, bc_ref, dec_ref):
    s = jnp.sum(x_ref[0].reshape(C_IN, H * W_DIM), axis=1)
    mean = s * (1.0 / (H * W_DIM))
    logits = jnp.sum(wc_ref[...] * mean[None, :], axis=1) + bc_ref[...]
    maxv = jnp.max(logits)
    idx = lax.broadcasted_iota(jnp.int32, (E,), 0)
    dec = jnp.min(jnp.where(logits == maxv, idx, E + 1))
    dec_ref[pl.program_id(0)] = dec


def _conv_body(sd_ref, od_ref, x_ref, w_ref, b_ref, o_ref):
    col = lax.broadcasted_iota(jnp.int32, (1, NPIX), 1) % W_DIM
    mask_m = (col != 0).astype(jnp.float32)
    mask_p = (col != W_DIM - 1).astype(jnp.float32)
    acc = jnp.zeros((C_OUT, NPIX), dtype=jnp.float32)
    for k in range(9):
        kh, kw = k // 3, k % 3
        s = (kh - 1) * W_DIM + (kw - 1)
        xs = x_ref[0, :, pl.ds(LPAD + s, NPIX)]
        contrib = lax.dot_general(
            w_ref[0, k], xs,
            (((1,), (0,)), ((), ())),
            preferred_element_type=jnp.float32,
        )
        if kw == 0:
            contrib = contrib * mask_m
        elif kw == 2:
            contrib = contrib * mask_p
        acc = acc + contrib
    o_ref[0] = acc + b_ref[0, 0, :][:, None]


def _sc_dispatch(dec_hbm, sdec_hbm, ord_hbm, dec_v, sdec_v, ord_v):
    wid = lax.axis_index("s") * 2 + lax.axis_index("c")

    @pl.when(wid == 0)
    def _():
        pltpu.sync_copy(dec_hbm, dec_v)
        keys = dec_v[...]
        lanes = lax.iota(jnp.int32, B)
        sd, so = plsc.sort_key_val(keys, lanes)
        sdec_v[...] = sd
        ord_v[...] = so
        pltpu.sync_copy(sdec_v, sdec_hbm)
        pltpu.sync_copy(ord_v, ord_hbm)


def kernel(x, W_ctl, b_ctl, W_exp, b_exp):
    decisions = pl.pallas_call(
        _router_body,
        grid=(B,),
        in_specs=[
            pl.BlockSpec((1, C_IN, H, W_DIM), lambda b: (b, 0, 0, 0)),
            pl.BlockSpec((E, C_IN), lambda b: (0, 0)),
            pl.BlockSpec((E,), lambda b: (0,)),
        ],
        out_specs=pl.BlockSpec((B,), lambda b: (0,), memory_space=pltpu.SMEM),
        out_shape=jax.ShapeDtypeStruct((B,), jnp.int32),
    )(x, W_ctl[:, :, 0, 0], b_ctl)

    mesh = plsc.VectorSubcoreMesh(core_axis_name="c", subcore_axis_name="s")
    sorted_dec, order = pl.kernel(
        _sc_dispatch,
        mesh=mesh,
        out_type=[
            jax.ShapeDtypeStruct((B,), jnp.int32),
            jax.ShapeDtypeStruct((B,), jnp.int32),
        ],
        scratch_types=[
            pltpu.VMEM((B,), jnp.int32),
            pltpu.VMEM((B,), jnp.int32),
            pltpu.VMEM((B,), jnp.int32),
        ],
        compiler_params=pltpu.CompilerParams(needs_layout_passes=False),
    )(decisions)

    x_p = jnp.pad(x.reshape(B, C_IN, NPIX).astype(jnp.bfloat16),
                  ((0, 0), (0, 0), (LPAD, LPAD)))
    w_r = W_exp.astype(jnp.bfloat16).transpose(0, 3, 4, 1, 2).reshape(E, 9, C_OUT, C_IN)

    grid_spec = pltpu.PrefetchScalarGridSpec(
        num_scalar_prefetch=2,
        grid=(B,),
        in_specs=[
            pl.BlockSpec((1, C_IN, XCOLS), lambda b, sd, od: (od[b], 0, 0)),
            pl.BlockSpec((1, 9, C_OUT, C_IN), lambda b, sd, od: (sd[b], 0, 0, 0)),
            pl.BlockSpec((1, 1, C_OUT), lambda b, sd, od: (sd[b], 0, 0)),
        ],
        out_specs=pl.BlockSpec((1, C_OUT, NPIX), lambda b, sd, od: (od[b], 0, 0)),
    )
    out = pl.pallas_call(
        _conv_body,
        grid_spec=grid_spec,
        out_shape=jax.ShapeDtypeStruct((B, C_OUT, NPIX), jnp.float32),
    )(sorted_dec, order, x_p, w_r, b_exp.reshape(E, 1, C_OUT))

    return out.reshape(B, C_OUT, H, W_DIM)


# speedup vs baseline: 4.4234x; 1.3801x over previous
import functools

import jax
import jax.numpy as jnp
from jax import lax
from jax.experimental import pallas as pl
from jax.experimental.pallas import tpu as pltpu
from jax.experimental.pallas import tpu_sc as plsc

B, C_IN, C_OUT, H, W_DIM, E, K = 16, 384, 384, 56, 56, 8, 3
NPIX = H * W_DIM
LPAD = 64
XCOLS = NPIX + 2 * LPAD


def _router_body(x_ref, wc_ref, bc_ref, xp_ref, dec_ref):
    xb = x_ref[0].astype(jnp.bfloat16)
    xp_ref[0, :, :LPAD] = jnp.zeros((C_IN, LPAD), jnp.bfloat16)
    xp_ref[0, :, pl.ds(LPAD, NPIX)] = xb
    xp_ref[0, :, pl.ds(LPAD + NPIX, LPAD)] = jnp.zeros((C_IN, LPAD), jnp.bfloat16)
    ones_p = jnp.ones((NPIX, 128), jnp.bfloat16)
    t = lax.dot_general(xb, ones_p, (((1,), (0,)), ((), ())),
                        preferred_element_type=jnp.float32)
    mean = t * (1.0 / (H * W_DIM))
    logits = lax.dot_general(wc_ref[...], mean, (((1,), (0,)), ((), ())),
                             preferred_element_type=jnp.float32)
    logits = logits + bc_ref[...][:, None]
    maxv = jnp.max(logits[:, :1])
    idx = lax.broadcasted_iota(jnp.int32, (E, 1), 0)
    dec = jnp.min(jnp.where(logits[:, :1] == maxv, idx, E + 1))
    dec_ref[pl.program_id(0)] = dec


def _conv_body(sd_ref, od_ref, x_ref, w_ref, b_ref, o_ref):
    col = lax.broadcasted_iota(jnp.int32, (1, NPIX), 1) % W_DIM
    mask_m = (col != 0).astype(jnp.float32)
    mask_p = (col != W_DIM - 1).astype(jnp.float32)
    acc = jnp.zeros((C_OUT, NPIX), dtype=jnp.float32)
    for k in range(9):
        kh, kw = k // 3, k % 3
        s = (kh - 1) * W_DIM + (kw - 1)
        xs = x_ref[0, :, pl.ds(LPAD + s, NPIX)]
        contrib = lax.dot_general(
            w_ref[0, k], xs,
            (((1,), (0,)), ((), ())),
            preferred_element_type=jnp.float32,
        )
        if kw == 0:
            contrib = contrib * mask_m
        elif kw == 2:
            contrib = contrib * mask_p
        acc = acc + contrib
    o_ref[0] = acc + b_ref[0, 0, :][:, None]


def _sc_dispatch(dec_hbm, sdec_hbm, ord_hbm, dec_v, sdec_v, ord_v):
    wid = lax.axis_index("s") * 2 + lax.axis_index("c")

    @pl.when(wid == 0)
    def _():
        pltpu.sync_copy(dec_hbm, dec_v)
        keys = dec_v[...]
        lanes = lax.iota(jnp.int32, B)
        sd, so = plsc.sort_key_val(keys, lanes)
        sdec_v[...] = sd
        ord_v[...] = so
        pltpu.sync_copy(sdec_v, sdec_hbm)
        pltpu.sync_copy(ord_v, ord_hbm)


def kernel(x, W_ctl, b_ctl, W_exp, b_exp):
    x_p, decisions = pl.pallas_call(
        _router_body,
        grid=(B,),
        in_specs=[
            pl.BlockSpec((1, C_IN, NPIX), lambda b: (b, 0, 0)),
            pl.BlockSpec((E, C_IN), lambda b: (0, 0)),
            pl.BlockSpec((E,), lambda b: (0,)),
        ],
        out_specs=[
            pl.BlockSpec((1, C_IN, XCOLS), lambda b: (b, 0, 0)),
            pl.BlockSpec((B,), lambda b: (0,), memory_space=pltpu.SMEM),
        ],
        out_shape=[
            jax.ShapeDtypeStruct((B, C_IN, XCOLS), jnp.bfloat16),
            jax.ShapeDtypeStruct((B,), jnp.int32),
        ],
    )(x.reshape(B, C_IN, NPIX), W_ctl[:, :, 0, 0], b_ctl)

    mesh = plsc.VectorSubcoreMesh(core_axis_name="c", subcore_axis_name="s")
    sorted_dec, order = pl.kernel(
        _sc_dispatch,
        mesh=mesh,
        out_type=[
            jax.ShapeDtypeStruct((B,), jnp.int32),
            jax.ShapeDtypeStruct((B,), jnp.int32),
        ],
        scratch_types=[
            pltpu.VMEM((B,), jnp.int32),
            pltpu.VMEM((B,), jnp.int32),
            pltpu.VMEM((B,), jnp.int32),
        ],
        compiler_params=pltpu.CompilerParams(needs_layout_passes=False),
    )(decisions)

    w_r = W_exp.astype(jnp.bfloat16).transpose(0, 3, 4, 1, 2).reshape(E, 9, C_OUT, C_IN)

    grid_spec = pltpu.PrefetchScalarGridSpec(
        num_scalar_prefetch=2,
        grid=(B,),
        in_specs=[
            pl.BlockSpec((1, C_IN, XCOLS), lambda b, sd, od: (od[b], 0, 0)),
            pl.BlockSpec((1, 9, C_OUT, C_IN), lambda b, sd, od: (sd[b], 0, 0, 0)),
            pl.BlockSpec((1, 1, C_OUT), lambda b, sd, od: (sd[b], 0, 0)),
        ],
        out_specs=pl.BlockSpec((1, C_OUT, NPIX), lambda b, sd, od: (od[b], 0, 0)),
    )
    out = pl.pallas_call(
        _conv_body,
        grid_spec=grid_spec,
        out_shape=jax.ShapeDtypeStruct((B, C_OUT, NPIX), jnp.float32),
    )(sorted_dec, order, x_p, w_r, b_exp.reshape(E, 1, C_OUT))

    return out.reshape(B, C_OUT, H, W_DIM)


# speedup vs baseline: 5.7552x; 1.3011x over previous
import jax
import jax.numpy as jnp
from jax import lax
from jax.experimental import pallas as pl
from jax.experimental.pallas import tpu as pltpu
from jax.experimental.pallas import tpu_sc as plsc

B, C_IN, C_OUT, H, W_DIM, E, K = 16, 384, 384, 56, 56, 8, 3
NPIX = H * W_DIM
RPAD = 64
XROWS = NPIX + 2 * RPAD


def _sc_dispatch(dec_hbm, sdec_hbm, ord_hbm, dec_v, sdec_v, ord_v):
    wid = lax.axis_index("s") * 2 + lax.axis_index("c")

    @pl.when(wid == 0)
    def _():
        pltpu.sync_copy(dec_hbm, dec_v)
        keys = dec_v[...]
        lanes = lax.iota(jnp.int32, B)
        sd, so = plsc.sort_key_val(keys, lanes)
        sdec_v[...] = sd
        ord_v[...] = so
        pltpu.sync_copy(sdec_v, sdec_hbm)
        pltpu.sync_copy(ord_v, ord_hbm)


def _router_body(x_ref, wc_ref, bc_ref, xp_ref, dec_ref):
    xb = x_ref[0].astype(jnp.bfloat16)
    xp_ref[0, :RPAD, :] = jnp.zeros((RPAD, C_IN), jnp.bfloat16)
    xp_ref[0, pl.ds(RPAD, NPIX), :] = xb
    xp_ref[0, pl.ds(RPAD + NPIX, RPAD), :] = jnp.zeros((RPAD, C_IN), jnp.bfloat16)
    ones_p = jnp.ones((E, NPIX), jnp.bfloat16)
    t = lax.dot_general(ones_p, xb, (((1,), (0,)), ((), ())),
                        preferred_element_type=jnp.float32)
    logits = jnp.sum(wc_ref[...] * t, axis=1) * (1.0 / (H * W_DIM)) + bc_ref[...]
    maxv = jnp.max(logits)
    idx = lax.broadcasted_iota(jnp.int32, (E,), 0)
    dec = jnp.min(jnp.where(logits == maxv, idx, E + 1))
    dec_ref[pl.program_id(0)] = dec


def _conv_body(sd_ref, od_ref, x_ref, w_ref, b_ref, o_ref, xcat_ref):
    row = lax.broadcasted_iota(jnp.int32, (NPIX, 1), 0) % W_DIM
    mask_m = (row != 0).astype(jnp.bfloat16)
    mask_p = (row != W_DIM - 1).astype(jnp.bfloat16)
    for k in range(9):
        kh, kw = k // 3, k % 3
        s = (kh - 1) * W_DIM + (kw - 1)
        xs = x_ref[0, pl.ds(RPAD + s, NPIX), :]
        if kw == 0:
            xs = xs * mask_m
        elif kw == 2:
            xs = xs * mask_p
        xcat_ref[:, k * C_IN:(k + 1) * C_IN] = xs
    wcat = w_ref[0].reshape(9 * C_IN, C_OUT)
    acc = lax.dot_general(
        xcat_ref[...], wcat,
        (((1,), (0,)), ((), ())),
        preferred_element_type=jnp.float32,
    )
    o_ref[0] = acc + b_ref[0, 0, :]


def kernel(x, W_ctl, b_ctl, W_exp, b_exp):
    x_p, decisions = pl.pallas_call(
        _router_body,
        grid=(B,),
        in_specs=[
            pl.BlockSpec((1, NPIX, C_IN), lambda b: (b, 0, 0)),
            pl.BlockSpec((E, C_IN), lambda b: (0, 0)),
            pl.BlockSpec((E,), lambda b: (0,)),
        ],
        out_specs=[
            pl.BlockSpec((1, XROWS, C_IN), lambda b: (b, 0, 0)),
            pl.BlockSpec((B,), lambda b: (0,), memory_space=pltpu.SMEM),
        ],
        out_shape=[
            jax.ShapeDtypeStruct((B, XROWS, C_IN), jnp.bfloat16),
            jax.ShapeDtypeStruct((B,), jnp.int32),
        ],
    )(x.transpose(0, 2, 3, 1).reshape(B, NPIX, C_IN), W_ctl[:, :, 0, 0], b_ctl)

    mesh = plsc.VectorSubcoreMesh(core_axis_name="c", subcore_axis_name="s")
    sorted_dec, order = pl.kernel(
        _sc_dispatch,
        mesh=mesh,
        out_type=[
            jax.ShapeDtypeStruct((B,), jnp.int32),
            jax.ShapeDtypeStruct((B,), jnp.int32),
        ],
        scratch_types=[
            pltpu.VMEM((B,), jnp.int32),
            pltpu.VMEM((B,), jnp.int32),
            pltpu.VMEM((B,), jnp.int32),
        ],
        compiler_params=pltpu.CompilerParams(needs_layout_passes=False),
    )(decisions)

    w_r = W_exp.astype(jnp.bfloat16).transpose(0, 3, 4, 2, 1).reshape(E, 9, C_IN, C_OUT)

    grid_spec = pltpu.PrefetchScalarGridSpec(
        num_scalar_prefetch=2,
        grid=(B,),
        in_specs=[
            pl.BlockSpec((1, XROWS, C_IN), lambda b, sd, od: (od[b], 0, 0)),
            pl.BlockSpec((1, 9, C_IN, C_OUT), lambda b, sd, od: (sd[b], 0, 0, 0)),
            pl.BlockSpec((1, 1, C_OUT), lambda b, sd, od: (sd[b], 0, 0)),
        ],
        out_specs=pl.BlockSpec((1, NPIX, C_OUT), lambda b, sd, od: (od[b], 0, 0)),
        scratch_shapes=[pltpu.VMEM((NPIX, 9 * C_IN), jnp.bfloat16)],
    )
    out = pl.pallas_call(
        _conv_body,
        grid_spec=grid_spec,
        out_shape=jax.ShapeDtypeStruct((B, NPIX, C_OUT), jnp.float32),
    )(sorted_dec, order, x_p, w_r, b_exp.reshape(E, 1, C_OUT))

    return out.reshape(B, H, W_DIM, C_OUT).transpose(0, 3, 1, 2)
